# baseline (speedup 1.0000x reference)
import jax
import jax.numpy as jnp
from jax.experimental import pallas as pl

_RADIUS2 = 64.0
_TI = 512


def _global_body(pos_ref, feat_ref, gf_ref, wgp_ref, wgf_ref, wgg_ref, bg_ref,
                 wlg_ref, bl_ref, gout_ref, gterm_ref):
    agg_p = jnp.max(pos_ref[...], axis=2)
    agg_f = jnp.max(feat_ref[...], axis=2)
    g_lin = (jnp.dot(agg_p, wgp_ref[...], preferred_element_type=jnp.float32)
             + jnp.dot(agg_f, wgf_ref[...], preferred_element_type=jnp.float32)
             + jnp.dot(gf_ref[...], wgg_ref[...], preferred_element_type=jnp.float32)
             + bg_ref[...])
    g_out = jnp.tanh(g_lin)
    gout_ref[...] = g_out
    G = wlg_ref.shape[0]
    gu = g_out[:, G:]
    gterm_ref[...] = (jnp.dot(gu, wlg_ref[...], preferred_element_type=jnp.float32)
                      + bl_ref[...])


def _local_body(pos_ref, rows_ref, nodes_ref, nodesr_ref, gterm_ref,
                wla_ref, wlb_ref, out_ref):
    cols = pos_ref[0]
    rows = rows_ref[0]
    c = wla_ref.shape[0]
    rn_c = jnp.sum(cols * cols, axis=0, keepdims=True)
    rn_r = jnp.sum(rows * rows, axis=1, keepdims=True)
    thresh = 0.5 * (rn_r - _RADIUS2) + 0.5 * rn_c
    dot = jnp.dot(rows, cols, preferred_element_type=jnp.float32,
                  precision=jax.lax.Precision.HIGHEST)
    mask = (dot > thresh).astype(jnp.float32)
    nsum = jnp.dot(mask, nodes_ref[0], preferred_element_type=jnp.float32)
    cnt = jnp.maximum(nsum[:, c:c + 1], 1.0)
    nmean = nsum[:, :c] / cnt
    lin = (jnp.dot(nodesr_ref[0], wla_ref[...], preferred_element_type=jnp.float32)
           + jnp.dot(nmean, wlb_ref[...], preferred_element_type=jnp.float32)
           + gterm_ref[0])
    out_ref[0] = jnp.tanh(lin)


def kernel(positions, features, global_features, W_g, b_g, W_l, b_l):
    B, P, N = positions.shape
    F = features.shape[1]
    G = global_features.shape[1]
    C = P + F
    G2 = 2 * G

    wgp = W_g[:P]
    wgf = W_g[P:C]
    wgg = W_g[C:]
    wla = W_l[:C]
    wlb = W_l[C:2 * C]
    wlg = W_l[2 * C:]
    bg2 = b_g.reshape(1, G2)
    bl2 = b_l.reshape(1, C)

    g_out, gterm = pl.pallas_call(
        _global_body,
        out_shape=(
            jax.ShapeDtypeStruct((B, G2), jnp.float32),
            jax.ShapeDtypeStruct((B, C), jnp.float32),
        ),
    )(positions, features, global_features, wgp, wgf, wgg, bg2, wlg, bl2)

    nodes = jnp.concatenate([positions, features], axis=1).transpose(0, 2, 1)
    ones = jnp.ones((B, N, 1), jnp.float32)
    nodes_ext = jnp.concatenate([nodes, ones], axis=2)
    xyz = positions.transpose(0, 2, 1)

    grid = (B, N // _TI)
    local_out = pl.pallas_call(
        _local_body,
        grid=grid,
        in_specs=[
            pl.BlockSpec((1, P, N), lambda b, i: (b, 0, 0)),
            pl.BlockSpec((1, _TI, P), lambda b, i: (b, i, 0)),
            pl.BlockSpec((1, N, C + 1), lambda b, i: (b, 0, 0)),
            pl.BlockSpec((1, _TI, C), lambda b, i: (b, i, 0)),
            pl.BlockSpec((1, 1, C), lambda b, i: (b, 0, 0)),
            pl.BlockSpec((C, C), lambda b, i: (0, 0)),
            pl.BlockSpec((C, C), lambda b, i: (0, 0)),
        ],
        out_specs=pl.BlockSpec((1, _TI, C), lambda b, i: (b, i, 0)),
        out_shape=jax.ShapeDtypeStruct((B, N, C), jnp.float32),
    )(positions, xyz, nodes_ext, nodes, gterm.reshape(B, 1, C), wla, wlb)

    positions_new = local_out[:, :, :P].transpose(0, 2, 1)
    features_new = local_out[:, :, P:].transpose(0, 2, 1)
    global_new = g_out[:, :G]
    return (positions_new, features_new, global_new)


# speedup vs baseline: 1.7048x; 1.7048x over previous
import jax
import jax.numpy as jnp
from jax.experimental import pallas as pl

_RADIUS2 = 64.0
_TI = 512


def _global_body(pos_ref, feat_ref, gf_ref, wgp_ref, wgf_ref, wgg_ref, bg_ref,
                 wlg_ref, bl_ref, gout_ref, gterm_ref):
    agg_p = jnp.max(pos_ref[...], axis=2)
    agg_f = jnp.max(feat_ref[...], axis=2)
    g_lin = (jnp.dot(agg_p, wgp_ref[...], preferred_element_type=jnp.float32)
             + jnp.dot(agg_f, wgf_ref[...], preferred_element_type=jnp.float32)
             + jnp.dot(gf_ref[...], wgg_ref[...], preferred_element_type=jnp.float32)
             + bg_ref[...])
    g_out = jnp.tanh(g_lin)
    gout_ref[...] = g_out
    G = wlg_ref.shape[0]
    gu = g_out[:, G:]
    gterm_ref[...] = (jnp.dot(gu, wlg_ref[...], preferred_element_type=jnp.float32)
                      + bl_ref[...])


def _local_body(pos_ref, rows_ref, nodes_ref, nodesr_ref, gterm_ref,
                wla_ref, wlb_ref, out_ref):
    cols = pos_ref[0]
    rows = rows_ref[0]
    c = wla_ref.shape[0]
    dx = rows[:, 0:1] - cols[0:1, :]
    dy = rows[:, 1:2] - cols[1:2, :]
    dz = rows[:, 2:3] - cols[2:3, :]
    d2 = dx * dx + dy * dy + dz * dz
    mask = (d2 < _RADIUS2).astype(jnp.bfloat16)
    nsum = jnp.dot(mask, nodes_ref[0], preferred_element_type=jnp.float32)
    cnt = jnp.maximum(nsum[:, c:c + 1], 1.0)
    nmean = nsum[:, :c] / cnt
    lin = (jnp.dot(nodesr_ref[0], wla_ref[...], preferred_element_type=jnp.float32)
           + jnp.dot(nmean, wlb_ref[...], preferred_element_type=jnp.float32)
           + gterm_ref[0])
    out_ref[0] = jnp.tanh(lin)


def kernel(positions, features, global_features, W_g, b_g, W_l, b_l):
    B, P, N = positions.shape
    F = features.shape[1]
    G = global_features.shape[1]
    C = P + F
    G2 = 2 * G

    wgp = W_g[:P]
    wgf = W_g[P:C]
    wgg = W_g[C:]
    wla = W_l[:C]
    wlb = W_l[C:2 * C]
    wlg = W_l[2 * C:]
    bg2 = b_g.reshape(1, G2)
    bl2 = b_l.reshape(1, C)

    g_out, gterm = pl.pallas_call(
        _global_body,
        out_shape=(
            jax.ShapeDtypeStruct((B, G2), jnp.float32),
            jax.ShapeDtypeStruct((B, C), jnp.float32),
        ),
    )(positions, features, global_features, wgp, wgf, wgg, bg2, wlg, bl2)

    nodes = jnp.concatenate([positions, features], axis=1).transpose(0, 2, 1)
    ones = jnp.ones((B, N, 1), jnp.float32)
    nodes_ext = jnp.concatenate([nodes, ones], axis=2).astype(jnp.bfloat16)
    xyz = positions.transpose(0, 2, 1)

    grid = (B, N // _TI)
    local_out = pl.pallas_call(
        _local_body,
        grid=grid,
        in_specs=[
            pl.BlockSpec((1, P, N), lambda b, i: (b, 0, 0)),
            pl.BlockSpec((1, _TI, P), lambda b, i: (b, i, 0)),
            pl.BlockSpec((1, N, C + 1), lambda b, i: (b, 0, 0)),
            pl.BlockSpec((1, _TI, C), lambda b, i: (b, i, 0)),
            pl.BlockSpec((1, 1, C), lambda b, i: (b, 0, 0)),
            pl.BlockSpec((C, C), lambda b, i: (0, 0)),
            pl.BlockSpec((C, C), lambda b, i: (0, 0)),
        ],
        out_specs=pl.BlockSpec((1, _TI, C), lambda b, i: (b, i, 0)),
        out_shape=jax.ShapeDtypeStruct((B, N, C), jnp.float32),
    )(positions, xyz, nodes_ext, nodes, gterm.reshape(B, 1, C), wla, wlb)

    positions_new = local_out[:, :, :P].transpose(0, 2, 1)
    features_new = local_out[:, :, P:].transpose(0, 2, 1)
    global_new = g_out[:, :G]
    return (positions_new, features_new, global_new)


# speedup vs baseline: 1.9960x; 1.1708x over previous
import jax
import jax.numpy as jnp
from jax.experimental import pallas as pl

_RADIUS2 = 64.0
_TI = 512


def _global_body(pos_ref, feat_ref, gf_ref, wgp_ref, wgf_ref, wgg_ref, bg_ref,
                 wlg_ref, bl_ref, gout_ref, gtp_ref, gtf_ref):
    agg_p = jnp.max(pos_ref[...], axis=2)
    agg_f = jnp.max(feat_ref[...], axis=2)
    g_lin = (jnp.dot(agg_p, wgp_ref[...], preferred_element_type=jnp.float32)
             + jnp.dot(agg_f, wgf_ref[...], preferred_element_type=jnp.float32)
             + jnp.dot(gf_ref[...], wgg_ref[...], preferred_element_type=jnp.float32)
             + bg_ref[...])
    g_out = jnp.tanh(g_lin)
    gout_ref[...] = g_out
    G = wlg_ref.shape[0]
    P = gtp_ref.shape[1]
    gu = g_out[:, G:]
    gterm = (jnp.dot(gu, wlg_ref[...], preferred_element_type=jnp.float32)
             + bl_ref[...])
    gtp_ref[...] = gterm[:, :P, None]
    gtf_ref[...] = gterm[:, P:, None]


def _local_body(xyzT_ref, posr_ref, featr_ref, pose_ref, featc_ref,
                gtp_ref, gtf_ref,
                app_ref, apf_ref, afp_ref, aff_ref,
                bpp_ref, bpf_ref, bfp_ref, bff_ref,
                outp_ref, outf_ref):
    xall = xyzT_ref[0, :, 0:1]
    yall = xyzT_ref[0, :, 1:2]
    zall = xyzT_ref[0, :, 2:3]
    xr = posr_ref[0, 0:1, :]
    yr = posr_ref[0, 1:2, :]
    zr = posr_ref[0, 2:3, :]
    dx = xall - xr
    dy = yall - yr
    dz = zall - zr
    d2 = dx * dx + dy * dy + dz * dz
    maskT = (d2 < _RADIUS2).astype(jnp.float32)
    a4 = jnp.dot(pose_ref[0], maskT, preferred_element_type=jnp.float32)
    sf = jnp.dot(featc_ref[0], maskT, preferred_element_type=jnp.float32)
    p = posr_ref.shape[1]
    cnt = jnp.maximum(a4[p:p + 1, :], 1.0)
    nmp = a4[:p, :] / cnt
    nmf = sf / cnt
    rp = posr_ref[0]
    rf = featr_ref[0]
    linp = (jnp.dot(app_ref[...], rp, preferred_element_type=jnp.float32)
            + jnp.dot(apf_ref[...], rf, preferred_element_type=jnp.float32)
            + jnp.dot(bpp_ref[...], nmp, preferred_element_type=jnp.float32)
            + jnp.dot(bpf_ref[...], nmf, preferred_element_type=jnp.float32)
            + gtp_ref[0])
    linf = (jnp.dot(afp_ref[...], rp, preferred_element_type=jnp.float32)
            + jnp.dot(aff_ref[...], rf, preferred_element_type=jnp.float32)
            + jnp.dot(bfp_ref[...], nmp, preferred_element_type=jnp.float32)
            + jnp.dot(bff_ref[...], nmf, preferred_element_type=jnp.float32)
            + gtf_ref[0])
    outp_ref[0] = jnp.tanh(linp)
    outf_ref[0] = jnp.tanh(linf)


def kernel(positions, features, global_features, W_g, b_g, W_l, b_l):
    B, P, N = positions.shape
    F = features.shape[1]
    G = global_features.shape[1]
    C = P + F
    G2 = 2 * G

    wgp = W_g[:P]
    wgf = W_g[P:C]
    wgg = W_g[C:]
    at = W_l[:C].T
    bt = W_l[C:2 * C].T
    wlg = W_l[2 * C:]
    bg2 = b_g.reshape(1, G2)
    bl2 = b_l.reshape(1, C)
    app, apf = at[:P, :P], at[:P, P:]
    afp, aff = at[P:, :P], at[P:, P:]
    bpp, bpf = bt[:P, :P], bt[:P, P:]
    bfp, bff = bt[P:, :P], bt[P:, P:]

    g_out, gtp, gtf = pl.pallas_call(
        _global_body,
        out_shape=(
            jax.ShapeDtypeStruct((B, G2), jnp.float32),
            jax.ShapeDtypeStruct((B, P, 1), jnp.float32),
            jax.ShapeDtypeStruct((B, F, 1), jnp.float32),
        ),
    )(positions, features, global_features, wgp, wgf, wgg, bg2, wlg, bl2)

    xyzT = positions.transpose(0, 2, 1)
    posext = jnp.concatenate(
        [positions, jnp.ones((B, 1, N), jnp.float32)], axis=1)

    grid = (B, N // _TI)
    wspec = pl.BlockSpec(None, lambda b, i: (0, 0))
    positions_new, features_new = pl.pallas_call(
        _local_body,
        grid=grid,
        in_specs=[
            pl.BlockSpec((1, N, P), lambda b, i: (b, 0, 0)),
            pl.BlockSpec((1, P, _TI), lambda b, i: (b, 0, i)),
            pl.BlockSpec((1, F, _TI), lambda b, i: (b, 0, i)),
            pl.BlockSpec((1, P + 1, N), lambda b, i: (b, 0, 0)),
            pl.BlockSpec((1, F, N), lambda b, i: (b, 0, 0)),
            pl.BlockSpec((1, P, 1), lambda b, i: (b, 0, 0)),
            pl.BlockSpec((1, F, 1), lambda b, i: (b, 0, 0)),
            pl.BlockSpec((P, P), lambda b, i: (0, 0)),
            pl.BlockSpec((P, F), lambda b, i: (0, 0)),
            pl.BlockSpec((F, P), lambda b, i: (0, 0)),
            pl.BlockSpec((F, F), lambda b, i: (0, 0)),
            pl.BlockSpec((P, P), lambda b, i: (0, 0)),
            pl.BlockSpec((P, F), lambda b, i: (0, 0)),
            pl.BlockSpec((F, P), lambda b, i: (0, 0)),
            pl.BlockSpec((F, F), lambda b, i: (0, 0)),
        ],
        out_specs=(
            pl.BlockSpec((1, P, _TI), lambda b, i: (b, 0, i)),
            pl.BlockSpec((1, F, _TI), lambda b, i: (b, 0, i)),
        ),
        out_shape=(
            jax.ShapeDtypeStruct((B, P, N), jnp.float32),
            jax.ShapeDtypeStruct((B, F, N), jnp.float32),
        ),
    )(xyzT, positions, features, posext, features, gtp, gtf,
      app, apf, afp, aff, bpp, bpf, bfp, bff)

    global_new = g_out[:, :G]
    return (positions_new, features_new, global_new)


# speedup vs baseline: 2.2308x; 1.1177x over previous
import jax
import jax.numpy as jnp
from jax.experimental import pallas as pl

_RADIUS2 = 64.0
_TI = 1024


def _global_body(pos_ref, feat_ref, gf_ref, wgp_ref, wgf_ref, wgg_ref, bg_ref,
                 wlg_ref, bl_ref, gout_ref, gtp_ref, gtf_ref):
    agg_p = jnp.max(pos_ref[...], axis=2)
    agg_f = jnp.max(feat_ref[...], axis=2)
    g_lin = (jnp.dot(agg_p, wgp_ref[...], preferred_element_type=jnp.float32)
             + jnp.dot(agg_f, wgf_ref[...], preferred_element_type=jnp.float32)
             + jnp.dot(gf_ref[...], wgg_ref[...], preferred_element_type=jnp.float32)
             + bg_ref[...])
    g_out = jnp.tanh(g_lin)
    gout_ref[...] = g_out
    G = wlg_ref.shape[0]
    P = gtp_ref.shape[1]
    gu = g_out[:, G:]
    gterm = (jnp.dot(gu, wlg_ref[...], preferred_element_type=jnp.float32)
             + bl_ref[...])
    gtp_ref[...] = gterm[:, :P, None]
    gtf_ref[...] = gterm[:, P:, None]


def _local_body(xyzT_ref, posr_ref, featr_ref, pose_ref, featc_ref,
                gtp_ref, gtf_ref,
                app_ref, apf_ref, afp_ref, aff_ref,
                bpp_ref, bpf_ref, bfp_ref, bff_ref,
                outp_ref, outf_ref):
    xall = xyzT_ref[0, :, 0:1]
    yall = xyzT_ref[0, :, 1:2]
    zall = xyzT_ref[0, :, 2:3]
    xr = posr_ref[0, 0:1, :]
    yr = posr_ref[0, 1:2, :]
    zr = posr_ref[0, 2:3, :]
    dx = xall - xr
    dy = yall - yr
    dz = zall - zr
    d2 = dx * dx + dy * dy + dz * dz
    maskT = (d2 < _RADIUS2).astype(jnp.float32)
    a4 = jnp.dot(pose_ref[0], maskT, preferred_element_type=jnp.float32)
    sf = jnp.dot(featc_ref[0], maskT, preferred_element_type=jnp.float32)
    p = posr_ref.shape[1]
    cnt = jnp.maximum(a4[p:p + 1, :], 1.0)
    nmp = a4[:p, :] / cnt
    nmf = sf / cnt
    rp = posr_ref[0]
    rf = featr_ref[0]
    linp = (jnp.dot(app_ref[...], rp, preferred_element_type=jnp.float32)
            + jnp.dot(apf_ref[...], rf, preferred_element_type=jnp.float32)
            + jnp.dot(bpp_ref[...], nmp, preferred_element_type=jnp.float32)
            + jnp.dot(bpf_ref[...], nmf, preferred_element_type=jnp.float32)
            + gtp_ref[0])
    linf = (jnp.dot(afp_ref[...], rp, preferred_element_type=jnp.float32)
            + jnp.dot(aff_ref[...], rf, preferred_element_type=jnp.float32)
            + jnp.dot(bfp_ref[...], nmp, preferred_element_type=jnp.float32)
            + jnp.dot(bff_ref[...], nmf, preferred_element_type=jnp.float32)
            + gtf_ref[0])
    outp_ref[0] = jnp.tanh(linp)
    outf_ref[0] = jnp.tanh(linf)


def kernel(positions, features, global_features, W_g, b_g, W_l, b_l):
    B, P, N = positions.shape
    F = features.shape[1]
    G = global_features.shape[1]
    C = P + F
    G2 = 2 * G

    wgp = W_g[:P]
    wgf = W_g[P:C]
    wgg = W_g[C:]
    at = W_l[:C].T
    bt = W_l[C:2 * C].T
    wlg = W_l[2 * C:]
    bg2 = b_g.reshape(1, G2)
    bl2 = b_l.reshape(1, C)
    app, apf = at[:P, :P], at[:P, P:]
    afp, aff = at[P:, :P], at[P:, P:]
    bpp, bpf = bt[:P, :P], bt[:P, P:]
    bfp, bff = bt[P:, :P], bt[P:, P:]

    g_out, gtp, gtf = pl.pallas_call(
        _global_body,
        out_shape=(
            jax.ShapeDtypeStruct((B, G2), jnp.float32),
            jax.ShapeDtypeStruct((B, P, 1), jnp.float32),
            jax.ShapeDtypeStruct((B, F, 1), jnp.float32),
        ),
    )(positions, features, global_features, wgp, wgf, wgg, bg2, wlg, bl2)

    xyzT = positions.transpose(0, 2, 1)
    posext = jnp.concatenate(
        [positions, jnp.ones((B, 1, N), jnp.float32)], axis=1)

    grid = (B, N // _TI)
    wspec = pl.BlockSpec(None, lambda b, i: (0, 0))
    positions_new, features_new = pl.pallas_call(
        _local_body,
        grid=grid,
        in_specs=[
            pl.BlockSpec((1, N, P), lambda b, i: (b, 0, 0)),
            pl.BlockSpec((1, P, _TI), lambda b, i: (b, 0, i)),
            pl.BlockSpec((1, F, _TI), lambda b, i: (b, 0, i)),
            pl.BlockSpec((1, P + 1, N), lambda b, i: (b, 0, 0)),
            pl.BlockSpec((1, F, N), lambda b, i: (b, 0, 0)),
            pl.BlockSpec((1, P, 1), lambda b, i: (b, 0, 0)),
            pl.BlockSpec((1, F, 1), lambda b, i: (b, 0, 0)),
            pl.BlockSpec((P, P), lambda b, i: (0, 0)),
            pl.BlockSpec((P, F), lambda b, i: (0, 0)),
            pl.BlockSpec((F, P), lambda b, i: (0, 0)),
            pl.BlockSpec((F, F), lambda b, i: (0, 0)),
            pl.BlockSpec((P, P), lambda b, i: (0, 0)),
            pl.BlockSpec((P, F), lambda b, i: (0, 0)),
            pl.BlockSpec((F, P), lambda b, i: (0, 0)),
            pl.BlockSpec((F, F), lambda b, i: (0, 0)),
        ],
        out_specs=(
            pl.BlockSpec((1, P, _TI), lambda b, i: (b, 0, i)),
            pl.BlockSpec((1, F, _TI), lambda b, i: (b, 0, i)),
        ),
        out_shape=(
            jax.ShapeDtypeStruct((B, P, N), jnp.float32),
            jax.ShapeDtypeStruct((B, F, N), jnp.float32),
        ),
    )(xyzT, positions, features, posext, features, gtp, gtf,
      app, apf, afp, aff, bpp, bpf, bfp, bff)

    global_new = g_out[:, :G]
    return (positions_new, features_new, global_new)


# speedup vs baseline: 2.3071x; 1.0342x over previous
import jax
import jax.numpy as jnp
from jax.experimental import pallas as pl

_RADIUS2 = 64.0
_TI = 2048


def _global_body(pos_ref, feat_ref, gf_ref, wgp_ref, wgf_ref, wgg_ref, bg_ref,
                 wlg_ref, bl_ref, gout_ref, gtp_ref, gtf_ref):
    agg_p = jnp.max(pos_ref[...], axis=2)
    agg_f = jnp.max(feat_ref[...], axis=2)
    g_lin = (jnp.dot(agg_p, wgp_ref[...], preferred_element_type=jnp.float32)
             + jnp.dot(agg_f, wgf_ref[...], preferred_element_type=jnp.float32)
             + jnp.dot(gf_ref[...], wgg_ref[...], preferred_element_type=jnp.float32)
             + bg_ref[...])
    g_out = jnp.tanh(g_lin)
    gout_ref[...] = g_out
    G = wlg_ref.shape[0]
    P = gtp_ref.shape[1]
    gu = g_out[:, G:]
    gterm = (jnp.dot(gu, wlg_ref[...], preferred_element_type=jnp.float32)
             + bl_ref[...])
    gtp_ref[...] = gterm[:, :P, None]
    gtf_ref[...] = gterm[:, P:, None]


def _local_body(xyzT_ref, posr_ref, featr_ref, pose_ref, featc_ref,
                gtp_ref, gtf_ref,
                app_ref, apf_ref, afp_ref, aff_ref,
                bpp_ref, bpf_ref, bfp_ref, bff_ref,
                outp_ref, outf_ref):
    xall = xyzT_ref[0, :, 0:1]
    yall = xyzT_ref[0, :, 1:2]
    zall = xyzT_ref[0, :, 2:3]
    xr = posr_ref[0, 0:1, :]
    yr = posr_ref[0, 1:2, :]
    zr = posr_ref[0, 2:3, :]
    dx = xall - xr
    dy = yall - yr
    dz = zall - zr
    d2 = dx * dx + dy * dy + dz * dz
    maskT = (d2 < _RADIUS2).astype(jnp.float32)
    a4 = jnp.dot(pose_ref[0], maskT, preferred_element_type=jnp.float32)
    sf = jnp.dot(featc_ref[0], maskT, preferred_element_type=jnp.float32)
    p = posr_ref.shape[1]
    cnt = jnp.maximum(a4[p:p + 1, :], 1.0)
    nmp = a4[:p, :] / cnt
    nmf = sf / cnt
    rp = posr_ref[0]
    rf = featr_ref[0]
    linp = (jnp.dot(app_ref[...], rp, preferred_element_type=jnp.float32)
            + jnp.dot(apf_ref[...], rf, preferred_element_type=jnp.float32)
            + jnp.dot(bpp_ref[...], nmp, preferred_element_type=jnp.float32)
            + jnp.dot(bpf_ref[...], nmf, preferred_element_type=jnp.float32)
            + gtp_ref[0])
    linf = (jnp.dot(afp_ref[...], rp, preferred_element_type=jnp.float32)
            + jnp.dot(aff_ref[...], rf, preferred_element_type=jnp.float32)
            + jnp.dot(bfp_ref[...], nmp, preferred_element_type=jnp.float32)
            + jnp.dot(bff_ref[...], nmf, preferred_element_type=jnp.float32)
            + gtf_ref[0])
    outp_ref[0] = jnp.tanh(linp)
    outf_ref[0] = jnp.tanh(linf)


def kernel(positions, features, global_features, W_g, b_g, W_l, b_l):
    B, P, N = positions.shape
    F = features.shape[1]
    G = global_features.shape[1]
    C = P + F
    G2 = 2 * G

    wgp = W_g[:P]
    wgf = W_g[P:C]
    wgg = W_g[C:]
    at = W_l[:C].T
    bt = W_l[C:2 * C].T
    wlg = W_l[2 * C:]
    bg2 = b_g.reshape(1, G2)
    bl2 = b_l.reshape(1, C)
    app, apf = at[:P, :P], at[:P, P:]
    afp, aff = at[P:, :P], at[P:, P:]
    bpp, bpf = bt[:P, :P], bt[:P, P:]
    bfp, bff = bt[P:, :P], bt[P:, P:]

    g_out, gtp, gtf = pl.pallas_call(
        _global_body,
        out_shape=(
            jax.ShapeDtypeStruct((B, G2), jnp.float32),
            jax.ShapeDtypeStruct((B, P, 1), jnp.float32),
            jax.ShapeDtypeStruct((B, F, 1), jnp.float32),
        ),
    )(positions, features, global_features, wgp, wgf, wgg, bg2, wlg, bl2)

    xyzT = positions.transpose(0, 2, 1)
    posext = jnp.concatenate(
        [positions, jnp.ones((B, 1, N), jnp.float32)], axis=1)

    grid = (B, N // _TI)
    wspec = pl.BlockSpec(None, lambda b, i: (0, 0))
    positions_new, features_new = pl.pallas_call(
        _local_body,
        grid=grid,
        in_specs=[
            pl.BlockSpec((1, N, P), lambda b, i: (b, 0, 0)),
            pl.BlockSpec((1, P, _TI), lambda b, i: (b, 0, i)),
            pl.BlockSpec((1, F, _TI), lambda b, i: (b, 0, i)),
            pl.BlockSpec((1, P + 1, N), lambda b, i: (b, 0, 0)),
            pl.BlockSpec((1, F, N), lambda b, i: (b, 0, 0)),
            pl.BlockSpec((1, P, 1), lambda b, i: (b, 0, 0)),
            pl.BlockSpec((1, F, 1), lambda b, i: (b, 0, 0)),
            pl.BlockSpec((P, P), lambda b, i: (0, 0)),
            pl.BlockSpec((P, F), lambda b, i: (0, 0)),
            pl.BlockSpec((F, P), lambda b, i: (0, 0)),
            pl.BlockSpec((F, F), lambda b, i: (0, 0)),
            pl.BlockSpec((P, P), lambda b, i: (0, 0)),
            pl.BlockSpec((P, F), lambda b, i: (0, 0)),
            pl.BlockSpec((F, P), lambda b, i: (0, 0)),
            pl.BlockSpec((F, F), lambda b, i: (0, 0)),
        ],
        out_specs=(
            pl.BlockSpec((1, P, _TI), lambda b, i: (b, 0, i)),
            pl.BlockSpec((1, F, _TI), lambda b, i: (b, 0, i)),
        ),
        out_shape=(
            jax.ShapeDtypeStruct((B, P, N), jnp.float32),
            jax.ShapeDtypeStruct((B, F, N), jnp.float32),
        ),
    )(xyzT, positions, features, posext, features, gtp, gtf,
      app, apf, afp, aff, bpp, bpf, bfp, bff)

    global_new = g_out[:, :G]
    return (positions_new, features_new, global_new)
